# Initial kernel scaffold; baseline (speedup 1.0000x reference)
import jax
import jax.numpy as jnp
from jax.experimental import pallas as pl


def kernel(x, edge_index, batch, W1, b1, g1, be1, W2, b2, g2, be2, fcW, fcb):
    raise NotImplementedError("write your pallas kernel here")


# speedup vs baseline: 8.9028x; 8.9028x over previous
import functools

import jax
import jax.numpy as jnp
from jax import lax
from jax.experimental import pallas as pl
from jax.experimental.pallas import tpu as pltpu
from jax.experimental.pallas import tpu_sc as plsc

N = 10000
E = 320000
D = 128
OUT = 128
G = 64

NC = 2
NS = 16
NW = NC * NS
L = 16

N_PAD = 10240
CHUNK = 128
E_PAD = 327680
ROWS = E_PAD // CHUNK
RPW = ROWS // NW
ACC_RPT = N_PAD // NS

_MESH = plsc.VectorSubcoreMesh(
    core_axis_name="c", subcore_axis_name="s", num_cores=NC, num_subcores=NS)


DEGW = 16

@functools.partial(
    pl.kernel,
    out_type=jax.ShapeDtypeStruct((NC, N_PAD, DEGW), jnp.float32),
    mesh=_MESH,
    scratch_types=[
        pltpu.VMEM((RPW, CHUNK), jnp.int32),
        pltpu.VMEM((CHUNK, DEGW), jnp.float32),
        pltpu.VMEM((L, DEGW), jnp.float32),
        pltpu.VMEM_SHARED((N_PAD, DEGW), jnp.float32),
    ],
)
def _deg_kernel(dst_hbm, deg_hbm, didx_v, ones_v, zb_v, acc_sh):
    cid = lax.axis_index("c")
    sid = lax.axis_index("s")
    wid = sid * NC + cid
    zeros = jnp.zeros((L,), jnp.float32)
    ones = jnp.ones((L,), jnp.float32)

    for i in range(L):
        zb_v[i, pl.ds(0, L)] = zeros
    for i in range(CHUNK):
        ones_v[i, pl.ds(0, L)] = ones

    abase = sid * ACC_RPT
    for t in range(ACC_RPT // L):
        pltpu.sync_copy(zb_v, acc_sh.at[pl.ds(pl.multiple_of(abase + t * L, 8), L)])

    rbase = pl.multiple_of(wid * RPW, 8)
    pltpu.sync_copy(dst_hbm.at[pl.ds(rbase, RPW)], didx_v)
    plsc.subcore_barrier()

    def body(j, _):
        pltpu.sync_copy(ones_v, acc_sh.at[didx_v.at[j]], add=True)
        return 0
    lax.fori_loop(0, RPW, body, 0)

    plsc.subcore_barrier()
    obase = pl.multiple_of(abase, 8)
    pltpu.sync_copy(acc_sh.at[pl.ds(obase, ACC_RPT)],
                    deg_hbm.at[cid, pl.ds(obase, ACC_RPT)])


@functools.partial(
    pl.kernel,
    out_type=jax.ShapeDtypeStruct((NC, N_PAD, D), jnp.float32),
    mesh=_MESH,
    scratch_types=[
        pltpu.VMEM((RPW, CHUNK), jnp.int32),
        pltpu.VMEM((RPW, CHUNK), jnp.int32),
        pltpu.VMEM((CHUNK, D), jnp.float32),
        pltpu.VMEM((L, D), jnp.float32),
        pltpu.VMEM_SHARED((N_PAD, D), jnp.float32),
        pltpu.SemaphoreType.DMA,
    ],
)
def _msg_kernel(y_hbm, src_hbm, dst_hbm, out_hbm,
                sidx_v, didx_v, rows_v, zb_v, acc_sh, sem):
    cid = lax.axis_index("c")
    sid = lax.axis_index("s")
    wid = sid * NC + cid
    zeros = jnp.zeros((L,), jnp.float32)

    for i in range(L):
        for j in range(D // L):
            zb_v[i, pl.ds(j * L, L)] = zeros
    abase = sid * ACC_RPT
    for t in range(ACC_RPT // L):
        pltpu.sync_copy(zb_v, acc_sh.at[pl.ds(pl.multiple_of(abase + t * L, 8), L)])

    rbase = pl.multiple_of(wid * RPW, 8)
    pltpu.sync_copy(src_hbm.at[pl.ds(rbase, RPW)], sidx_v)
    pltpu.sync_copy(dst_hbm.at[pl.ds(rbase, RPW)], didx_v)
    plsc.subcore_barrier()

    def body(j, _):
        pltpu.async_copy(y_hbm.at[sidx_v.at[j]], rows_v, sem).wait()
        pltpu.sync_copy(rows_v, acc_sh.at[didx_v.at[j]], add=True)
        return 0
    lax.fori_loop(0, RPW, body, 0)

    plsc.subcore_barrier()
    obase = pl.multiple_of(abase, 8)
    pltpu.sync_copy(acc_sh.at[pl.ds(obase, ACC_RPT)],
                    out_hbm.at[cid, pl.ds(obase, ACC_RPT)])


def _tc_pre_body(x_ref, w1_ref, degp_ref, y1_ref, dinv_ref):
    deg = degp_ref[0] + degp_ref[1]
    dinv = lax.rsqrt(deg + 1.0)
    xw = jnp.dot(x_ref[...], w1_ref[...], preferred_element_type=jnp.float32)
    y1_ref[...] = xw * dinv
    dinv_ref[...] = dinv


_tc_pre = pl.pallas_call(
    _tc_pre_body,
    out_shape=[jax.ShapeDtypeStruct((N, D), jnp.float32),
               jax.ShapeDtypeStruct((N, 1), jnp.float32)],
)


def _bn_relu(pre, g, be):
    mu = jnp.mean(pre, axis=0, keepdims=True)
    var = jnp.mean((pre - mu) * (pre - mu), axis=0, keepdims=True)
    return jnp.maximum((pre - mu) * lax.rsqrt(var + 1e-5) * g + be, 0.0)


def _tc_mid_body(part_ref, y1_ref, dinv_ref, b1_ref, g1_ref, be1_ref, w2_ref,
                 y2_ref):
    s = part_ref[0, :N, :] + part_ref[1, :N, :] + y1_ref[...]
    pre = s * dinv_ref[...] + b1_ref[...]
    h = _bn_relu(pre, g1_ref[...], be1_ref[...])
    y2_ref[...] = jnp.dot(h, w2_ref[...],
                          preferred_element_type=jnp.float32) * dinv_ref[...]


_tc_mid = pl.pallas_call(
    _tc_mid_body,
    out_shape=jax.ShapeDtypeStruct((N, D), jnp.float32),
)


def _tc_post_body(part_ref, y2_ref, dinv_ref, b2_ref, g2_ref, be2_ref,
                  batch_ref, fcw_ref, fcb_ref, out_ref):
    s = part_ref[0, :N, :] + part_ref[1, :N, :] + y2_ref[...]
    pre = s * dinv_ref[...] + b2_ref[...]
    h = _bn_relu(pre, g2_ref[...], be2_ref[...])
    gid = lax.broadcasted_iota(jnp.int32, (G, N), 0)
    oh = (batch_ref[...] == gid).astype(jnp.float32)
    pooled = jnp.dot(oh, h, preferred_element_type=jnp.float32)
    counts = jnp.sum(oh, axis=1, keepdims=True)
    pooled = pooled / jnp.maximum(counts, 1.0)
    out_ref[...] = jnp.dot(pooled, fcw_ref[...],
                           preferred_element_type=jnp.float32) + fcb_ref[...]


_tc_post = pl.pallas_call(
    _tc_post_body,
    out_shape=jax.ShapeDtypeStruct((G, OUT), jnp.float32),
)


def kernel(x, edge_index, batch, W1, b1, g1, be1, W2, b2, g2, be2, fcW, fcb):
    src = edge_index[0].astype(jnp.int32)
    dst = edge_index[1].astype(jnp.int32)
    pad = E_PAD - E
    src_p = jnp.concatenate([src, jnp.zeros((pad,), jnp.int32)])
    dst_p = jnp.concatenate([dst, jnp.full((pad,), N, jnp.int32)])
    src2d = src_p.reshape(ROWS, CHUNK)
    dst2d = dst_p.reshape(ROWS, CHUNK)

    degp = _deg_kernel(dst2d)
    deg_col = degp[:, :N, 0:1]

    y1, dinv = _tc_pre(x, W1, deg_col)
    part1 = _msg_kernel(y1, src2d, dst2d)
    y2 = _tc_mid(part1, y1, dinv,
                 b1.reshape(1, D), g1.reshape(1, D), be1.reshape(1, D), W2)
    part2 = _msg_kernel(y2, src2d, dst2d)
    return _tc_post(part2, y2, dinv,
                    b2.reshape(1, D), g2.reshape(1, D), be2.reshape(1, D),
                    batch.astype(jnp.int32).reshape(1, N), fcW,
                    fcb.reshape(1, OUT))


# speedup vs baseline: 9.9977x; 1.1230x over previous
import functools

import jax
import jax.numpy as jnp
from jax import lax
from jax.experimental import pallas as pl
from jax.experimental.pallas import tpu as pltpu
from jax.experimental.pallas import tpu_sc as plsc

N = 10000
E = 320000
D = 128
OUT = 128
G = 64

NC = 2
NS = 16
NW = NC * NS
L = 16

N_PAD = 10240
CHUNK = 128
E_PAD = 327680
ROWS = E_PAD // CHUNK
RPW = ROWS // NW
ACC_RPT = N_PAD // NS

_MESH = plsc.VectorSubcoreMesh(
    core_axis_name="c", subcore_axis_name="s", num_cores=NC, num_subcores=NS)


DEGW = 16

@functools.partial(
    pl.kernel,
    out_type=jax.ShapeDtypeStruct((NC, N_PAD, DEGW), jnp.float32),
    mesh=_MESH,
    scratch_types=[
        pltpu.VMEM((RPW, CHUNK), jnp.int32),
        pltpu.VMEM((CHUNK, DEGW), jnp.float32),
        pltpu.VMEM_SHARED((N_PAD, DEGW), jnp.float32),
        pltpu.SemaphoreType.DMA,
    ],
)
def _deg_kernel(dst_hbm, deg_hbm, didx_v, ones_v, acc_sh, sem):
    cid = lax.axis_index("c")
    sid = lax.axis_index("s")
    wid = sid * NC + cid
    zeros = jnp.zeros((L,), jnp.float32)
    ones = jnp.ones((L,), jnp.float32)

    def zb(i, _):
        ones_v[i, pl.ds(0, L)] = zeros
        return 0
    lax.fori_loop(0, CHUNK, zb, 0)
    abase = sid * ACC_RPT
    for t in range(ACC_RPT // CHUNK):
        pltpu.sync_copy(
            ones_v, acc_sh.at[pl.ds(pl.multiple_of(abase + t * CHUNK, 8), CHUNK)])

    def ob(i, _):
        ones_v[i, pl.ds(0, L)] = ones
        return 0
    lax.fori_loop(0, CHUNK, ob, 0)

    rbase = pl.multiple_of(wid * RPW, 8)
    pltpu.sync_copy(dst_hbm.at[pl.ds(rbase, RPW)], didx_v)
    plsc.subcore_barrier()

    FAN = 8
    def body(k, _):
        descs = [pltpu.async_copy(ones_v, acc_sh.at[didx_v.at[k * FAN + b]],
                                  sem, add=True)
                 for b in range(FAN)]
        for d in descs:
            d.wait()
        return 0
    lax.fori_loop(0, RPW // FAN, body, 0)

    plsc.subcore_barrier()
    obase = pl.multiple_of(abase, 8)
    pltpu.sync_copy(acc_sh.at[pl.ds(obase, ACC_RPT)],
                    deg_hbm.at[cid, pl.ds(obase, ACC_RPT)])


HALF = RPW // 2

@functools.partial(
    pl.kernel,
    out_type=jax.ShapeDtypeStruct((NC, N_PAD, D), jnp.float32),
    mesh=_MESH,
    scratch_types=[
        pltpu.VMEM((HALF, CHUNK), jnp.int32),
        pltpu.VMEM((HALF, CHUNK), jnp.int32),
        pltpu.VMEM((CHUNK, D), jnp.float32),
        pltpu.VMEM((CHUNK, D), jnp.float32),
        pltpu.VMEM_SHARED((N_PAD, D), jnp.float32),
        pltpu.SemaphoreType.DMA,
        pltpu.SemaphoreType.DMA,
    ],
)
def _msg_kernel(y_hbm, src_hbm, dst_hbm, out_hbm,
                sidx_v, didx_v, rows0_v, rows1_v, acc_sh, sem0, sem1):
    cid = lax.axis_index("c")
    sid = lax.axis_index("s")
    wid = sid * NC + cid
    zeros = jnp.zeros((L,), jnp.float32)

    def zb(i, _):
        for j in range(D // L):
            rows0_v[i, pl.ds(j * L, L)] = zeros
        return 0
    lax.fori_loop(0, CHUNK, zb, 0)
    abase = sid * ACC_RPT
    for t in range(ACC_RPT // CHUNK):
        pltpu.sync_copy(
            rows0_v, acc_sh.at[pl.ds(pl.multiple_of(abase + t * CHUNK, 8), CHUNK)])
    plsc.subcore_barrier()

    def wait0():
        pltpu.make_async_copy(y_hbm.at[sidx_v.at[0]], rows0_v, sem0).wait()

    def wait1():
        pltpu.make_async_copy(y_hbm.at[sidx_v.at[0]], rows1_v, sem1).wait()

    for h in range(RPW // HALF):
        hbase = pl.multiple_of(wid * RPW + h * HALF, 8)
        pltpu.sync_copy(src_hbm.at[pl.ds(hbase, HALF)], sidx_v)
        pltpu.sync_copy(dst_hbm.at[pl.ds(hbase, HALF)], didx_v)

        pltpu.async_copy(y_hbm.at[sidx_v.at[0]], rows0_v, sem0)

        def body(k, _):
            c = 2 * k
            pltpu.async_copy(y_hbm.at[sidx_v.at[c + 1]], rows1_v, sem1)
            wait0()
            pltpu.sync_copy(rows0_v, acc_sh.at[didx_v.at[c]], add=True)
            pltpu.async_copy(y_hbm.at[sidx_v.at[c + 2]], rows0_v, sem0)
            wait1()
            pltpu.sync_copy(rows1_v, acc_sh.at[didx_v.at[c + 1]], add=True)
            return 0
        lax.fori_loop(0, HALF // 2 - 1, body, 0)

        pltpu.async_copy(y_hbm.at[sidx_v.at[HALF - 1]], rows1_v, sem1)
        wait0()
        pltpu.sync_copy(rows0_v, acc_sh.at[didx_v.at[HALF - 2]], add=True)
        wait1()
        pltpu.sync_copy(rows1_v, acc_sh.at[didx_v.at[HALF - 1]], add=True)

    plsc.subcore_barrier()
    obase = pl.multiple_of(abase, 8)
    pltpu.sync_copy(acc_sh.at[pl.ds(obase, ACC_RPT)],
                    out_hbm.at[cid, pl.ds(obase, ACC_RPT)])


def _tc_pre_body(x_ref, w1_ref, degp_ref, y1_ref, dinv_ref):
    deg = degp_ref[0] + degp_ref[1]
    dinv = lax.rsqrt(deg + 1.0)
    xw = jnp.dot(x_ref[...], w1_ref[...], preferred_element_type=jnp.float32)
    y1_ref[...] = xw * dinv
    dinv_ref[...] = dinv


_tc_pre = pl.pallas_call(
    _tc_pre_body,
    out_shape=[jax.ShapeDtypeStruct((N, D), jnp.float32),
               jax.ShapeDtypeStruct((N, 1), jnp.float32)],
)


def _bn_relu(pre, g, be):
    mu = jnp.mean(pre, axis=0, keepdims=True)
    var = jnp.mean((pre - mu) * (pre - mu), axis=0, keepdims=True)
    return jnp.maximum((pre - mu) * lax.rsqrt(var + 1e-5) * g + be, 0.0)


def _tc_mid_body(part_ref, y1_ref, dinv_ref, b1_ref, g1_ref, be1_ref, w2_ref,
                 y2_ref):
    s = part_ref[0, :N, :] + part_ref[1, :N, :] + y1_ref[...]
    pre = s * dinv_ref[...] + b1_ref[...]
    h = _bn_relu(pre, g1_ref[...], be1_ref[...])
    y2_ref[...] = jnp.dot(h, w2_ref[...],
                          preferred_element_type=jnp.float32) * dinv_ref[...]


_tc_mid = pl.pallas_call(
    _tc_mid_body,
    out_shape=jax.ShapeDtypeStruct((N, D), jnp.float32),
)


def _tc_post_body(part_ref, y2_ref, dinv_ref, b2_ref, g2_ref, be2_ref,
                  batch_ref, fcw_ref, fcb_ref, out_ref):
    s = part_ref[0, :N, :] + part_ref[1, :N, :] + y2_ref[...]
    pre = s * dinv_ref[...] + b2_ref[...]
    h = _bn_relu(pre, g2_ref[...], be2_ref[...])
    gid = lax.broadcasted_iota(jnp.int32, (G, N), 0)
    oh = (batch_ref[...] == gid).astype(jnp.float32)
    pooled = jnp.dot(oh, h, preferred_element_type=jnp.float32)
    counts = jnp.sum(oh, axis=1, keepdims=True)
    pooled = pooled / jnp.maximum(counts, 1.0)
    out_ref[...] = jnp.dot(pooled, fcw_ref[...],
                           preferred_element_type=jnp.float32) + fcb_ref[...]


_tc_post = pl.pallas_call(
    _tc_post_body,
    out_shape=jax.ShapeDtypeStruct((G, OUT), jnp.float32),
)


def kernel(x, edge_index, batch, W1, b1, g1, be1, W2, b2, g2, be2, fcW, fcb):
    src = edge_index[0].astype(jnp.int32)
    dst = edge_index[1].astype(jnp.int32)
    pad = E_PAD - E
    src_p = jnp.concatenate([src, jnp.zeros((pad,), jnp.int32)])
    dst_p = jnp.concatenate([dst, jnp.full((pad,), N, jnp.int32)])
    src2d = src_p.reshape(ROWS, CHUNK)
    dst2d = dst_p.reshape(ROWS, CHUNK)

    degp = _deg_kernel(dst2d)
    deg_col = degp[:, :N, 0:1]

    y1, dinv = _tc_pre(x, W1, deg_col)
    part1 = _msg_kernel(y1, src2d, dst2d)
    y2 = _tc_mid(part1, y1, dinv,
                 b1.reshape(1, D), g1.reshape(1, D), be1.reshape(1, D), W2)
    part2 = _msg_kernel(y2, src2d, dst2d)
    return _tc_post(part2, y2, dinv,
                    b2.reshape(1, D), g2.reshape(1, D), be2.reshape(1, D),
                    batch.astype(jnp.int32).reshape(1, N), fcW,
                    fcb.reshape(1, OUT))
